# baseline (speedup 1.0000x reference)
import functools

import jax
import jax.numpy as jnp
from jax import lax
from jax.experimental import pallas as pl
from jax.experimental.pallas import tpu as pltpu
from jax.experimental.pallas import tpu_sc as plsc

F32 = jnp.float32

NC = 2
NS = 16
BLK = 128


def _proj_body(x_ref, wrel_ref, wroot_ref, t_ref, r_ref):
    x = x_ref[...]
    n = x.shape[0]
    p = jnp.dot(x, wrel_ref[...], preferred_element_type=F32)
    hi = (lax.broadcasted_iota(jnp.int32, (n, 8), 1) == 0).astype(F32)
    t_ref[...] = jnp.concatenate([p, hi], axis=1)
    r_ref[...] = jnp.dot(x, wroot_ref[...], preferred_element_type=F32)


def _layer1_body(agg_ref, r_ref, b_ref, wrel_ref, wroot_ref,
                 t_ref, rout_ref, degr_ref):
    nn = r_ref.shape[0]
    a = (agg_ref[0] + agg_ref[1])[:nn]
    degr_ref[...] = 1.0 / jnp.maximum(a[:, 8:9], 1.0)
    h = jnp.maximum(a[:, 0:8] + b_ref[...] + r_ref[...], 0.0)
    t_ref[...] = jnp.dot(h, wrel_ref[...], preferred_element_type=F32)
    rout_ref[...] = jnp.dot(h, wroot_ref[...], preferred_element_type=F32)


def _layer2_body(agg_ref, r_ref, degr_ref, b_ref, wrel_ref, wroot_ref,
                 t_ref, rout_ref):
    nn = r_ref.shape[0]
    a = (agg_ref[0] + agg_ref[1])[:nn]
    h = jnp.maximum(a[:, 0:8] * degr_ref[...] + b_ref[...] + r_ref[...], 0.0)
    t_ref[...] = jnp.dot(h, wrel_ref[...], preferred_element_type=F32)
    rout_ref[...] = jnp.dot(h, wroot_ref[...], preferred_element_type=F32)


def _final_body(agg_ref, r_ref, degr_ref, b_ref, batch_ref,
                wl1_ref, bl1_ref, wl2_ref, bl2_ref, out_ref):
    nn = r_ref.shape[0]
    a = (agg_ref[0] + agg_ref[1])[:nn]
    h = jnp.maximum(a[:, 0:8] * degr_ref[...] + b_ref[...] + r_ref[...], 0.0)
    n = h.shape[0]
    g = out_ref.shape[0]
    onehot = (batch_ref[...] ==
              lax.broadcasted_iota(jnp.int32, (g, n), 0)).astype(F32)
    pooled_sum = jnp.dot(onehot, h, preferred_element_type=F32)
    cnt = jnp.maximum(jnp.sum(onehot, axis=1, keepdims=True), 1.0)
    pooled = pooled_sum / cnt
    z = jnp.maximum(
        jnp.dot(pooled, wl1_ref[...], preferred_element_type=F32) + bl1_ref[...],
        0.0)
    out_ref[...] = (jnp.dot(z, wl2_ref[...], preferred_element_type=F32)
                    + bl2_ref[...])


_NBUF = 4


def _make_edge_pass(n, nblk, w):
    bpt = nblk // (NC * NS)
    nagg = (n // (NS * 8) + 1) * NS * 8
    zrows = nagg // NS
    mesh = plsc.VectorSubcoreMesh(core_axis_name="c", subcore_axis_name="s",
                                  num_cores=NC, num_subcores=NS)

    @functools.partial(
        pl.kernel,
        out_type=jax.ShapeDtypeStruct((NC, nagg, w), F32),
        mesh=mesh,
        scratch_types=[
            pltpu.VMEM((bpt, BLK), jnp.int32),
            pltpu.VMEM((bpt, BLK), jnp.int32),
            pltpu.VMEM((_NBUF, BLK, w), F32),
            pltpu.VMEM((zrows, w), F32),
            pltpu.VMEM_SHARED((nagg, w), F32),
            [pltpu.SemaphoreType.DMA] * _NBUF,
            [pltpu.SemaphoreType.DMA] * _NBUF,
        ],
        compiler_params=pltpu.CompilerParams(use_tc_tiling_on_sc=False),
    )
    def edge_pass(t_hbm, src_hbm, dst_hbm, z_hbm, agg_hbm,
                  src_v, dst_v, rows_v, zb_v, agg_sh, gsems, ssems):
        c = lax.axis_index("c")
        s = lax.axis_index("s")
        wid = c * NS + s
        pltpu.sync_copy(src_hbm.at[pl.ds(wid * bpt, bpt)], src_v)
        pltpu.sync_copy(dst_hbm.at[pl.ds(wid * bpt, bpt)], dst_v)
        pltpu.sync_copy(z_hbm, zb_v)
        pltpu.sync_copy(zb_v, agg_sh.at[pl.ds(s * zrows, zrows)])
        plsc.subcore_barrier()

        for j in range(_NBUF):
            pltpu.async_copy(t_hbm.at[src_v.at[j]], rows_v.at[j], gsems[j])

        def body(g, carry):
            base = g * _NBUF
            for j in range(_NBUF):
                b = base + j
                pltpu.make_async_copy(
                    t_hbm.at[src_v.at[b]], rows_v.at[j], gsems[j]).wait()
                pltpu.async_copy(rows_v.at[j], agg_sh.at[dst_v.at[b]],
                                 ssems[j], add=True)
            for j in range(_NBUF):
                b = base + j
                pltpu.make_async_copy(
                    rows_v.at[j], agg_sh.at[dst_v.at[b]], ssems[j]).wait()

                @pl.when(b + _NBUF < bpt)
                def _():
                    pltpu.async_copy(
                        t_hbm.at[src_v.at[b + _NBUF]], rows_v.at[j], gsems[j])
            return carry

        lax.fori_loop(0, bpt // _NBUF, body, 0)
        plsc.subcore_barrier()
        pltpu.sync_copy(agg_sh.at[pl.ds(s * zrows, zrows)],
                        agg_hbm.at[c, pl.ds(s * zrows, zrows)])

    return edge_pass


def kernel(x, edge_index, batch, W1_rel, b1, W1_root, W2_rel, b2, W2_root,
           W3_rel, b3, W3_root, W_lin1, b_lin1, W_lin2, b_lin2):
    n, d = x.shape
    e = edge_index.shape[1]
    h = W1_rel.shape[1]
    g = 64
    per_pass = BLK * NC * NS * 8
    epad = -(-e // per_pass) * per_pass
    nblk = epad // BLK
    trash = n

    src = edge_index[0]
    dst = edge_index[1]
    pad = epad - e
    nagg = (n // (NS * 8) + 1) * NS * 8
    pad_dst = trash + jnp.arange(pad, dtype=jnp.int32) % (nagg - n)
    srcb = jnp.concatenate([src, jnp.zeros((pad,), jnp.int32)]).reshape(nblk, BLK)
    dstb = jnp.concatenate([dst, pad_dst]).reshape(nblk, BLK)
    z16 = jnp.zeros((nagg // NS, 16), F32)
    z8 = jnp.zeros((nagg // NS, h), F32)

    edge_pass16 = _make_edge_pass(n, nblk, 16)
    edge_pass8 = _make_edge_pass(n, nblk, h)

    proj = pl.pallas_call(
        _proj_body,
        out_shape=[jax.ShapeDtypeStruct((n, 16), F32),
                   jax.ShapeDtypeStruct((n, h), F32)])
    layer1 = pl.pallas_call(
        _layer1_body,
        out_shape=[jax.ShapeDtypeStruct((n, h), F32),
                   jax.ShapeDtypeStruct((n, h), F32),
                   jax.ShapeDtypeStruct((n, 1), F32)])
    layer2 = pl.pallas_call(
        _layer2_body,
        out_shape=[jax.ShapeDtypeStruct((n, h), F32),
                   jax.ShapeDtypeStruct((n, h), F32)])
    final = pl.pallas_call(
        _final_body,
        out_shape=jax.ShapeDtypeStruct((g, 1), F32))

    t1, r1 = proj(x, W1_rel, W1_root)
    agg1 = edge_pass16(t1, srcb, dstb, z16)
    t2, r2, degr = layer1(agg1, r1, b1.reshape(1, h), W2_rel, W2_root)
    agg2 = edge_pass8(t2, srcb, dstb, z8)
    t3, r3 = layer2(agg2, r2, degr, b2.reshape(1, h), W3_rel, W3_root)
    agg3 = edge_pass8(t3, srcb, dstb, z8)
    out = final(agg3, r3, degr, b3.reshape(1, h),
                batch.reshape(1, n).astype(jnp.int32),
                W_lin1, b_lin1.reshape(1, -1), W_lin2, b_lin2.reshape(1, -1))
    return out


# speedup vs baseline: 1.0576x; 1.0576x over previous
import functools

import jax
import jax.numpy as jnp
from jax import lax
from jax.experimental import pallas as pl
from jax.experimental.pallas import tpu as pltpu
from jax.experimental.pallas import tpu_sc as plsc

F32 = jnp.float32

NC = 2
NS = 16
BLK = 256


def _proj_body(x_ref, wrel_ref, wroot_ref, t_ref, r_ref):
    x = x_ref[...]
    n = x.shape[0]
    p = jnp.dot(x, wrel_ref[...], preferred_element_type=F32)
    hi = (lax.broadcasted_iota(jnp.int32, (n, 8), 1) == 0).astype(F32)
    t_ref[...] = jnp.concatenate([p, hi], axis=1)
    r_ref[...] = jnp.dot(x, wroot_ref[...], preferred_element_type=F32)


def _layer1_body(agg_ref, r_ref, b_ref, wrel_ref, wroot_ref,
                 t_ref, rout_ref, degr_ref):
    nn = r_ref.shape[0]
    a = (agg_ref[0] + agg_ref[1])[:nn]
    degr_ref[...] = 1.0 / jnp.maximum(a[:, 8:9], 1.0)
    h = jnp.maximum(a[:, 0:8] + b_ref[...] + r_ref[...], 0.0)
    t_ref[...] = jnp.dot(h, wrel_ref[...], preferred_element_type=F32)
    rout_ref[...] = jnp.dot(h, wroot_ref[...], preferred_element_type=F32)


def _layer2_body(agg_ref, r_ref, degr_ref, b_ref, wrel_ref, wroot_ref,
                 t_ref, rout_ref):
    nn = r_ref.shape[0]
    a = (agg_ref[0] + agg_ref[1])[:nn]
    h = jnp.maximum(a[:, 0:8] * degr_ref[...] + b_ref[...] + r_ref[...], 0.0)
    t_ref[...] = jnp.dot(h, wrel_ref[...], preferred_element_type=F32)
    rout_ref[...] = jnp.dot(h, wroot_ref[...], preferred_element_type=F32)


def _final_body(agg_ref, r_ref, degr_ref, b_ref, batch_ref,
                wl1_ref, bl1_ref, wl2_ref, bl2_ref, out_ref):
    nn = r_ref.shape[0]
    a = (agg_ref[0] + agg_ref[1])[:nn]
    h = jnp.maximum(a[:, 0:8] * degr_ref[...] + b_ref[...] + r_ref[...], 0.0)
    n = h.shape[0]
    g = out_ref.shape[0]
    onehot = (batch_ref[...] ==
              lax.broadcasted_iota(jnp.int32, (g, n), 0)).astype(F32)
    pooled_sum = jnp.dot(onehot, h, preferred_element_type=F32)
    cnt = jnp.maximum(jnp.sum(onehot, axis=1, keepdims=True), 1.0)
    pooled = pooled_sum / cnt
    z = jnp.maximum(
        jnp.dot(pooled, wl1_ref[...], preferred_element_type=F32) + bl1_ref[...],
        0.0)
    out_ref[...] = (jnp.dot(z, wl2_ref[...], preferred_element_type=F32)
                    + bl2_ref[...])


_NBUF = 4


def _make_edge_pass(n, nblk, w):
    bpt = nblk // (NC * NS)
    nagg = (n // (NS * 8) + 1) * NS * 8
    zrows = nagg // NS
    mesh = plsc.VectorSubcoreMesh(core_axis_name="c", subcore_axis_name="s",
                                  num_cores=NC, num_subcores=NS)

    @functools.partial(
        pl.kernel,
        out_type=jax.ShapeDtypeStruct((NC, nagg, w), F32),
        mesh=mesh,
        scratch_types=[
            pltpu.VMEM((bpt, BLK), jnp.int32),
            pltpu.VMEM((bpt, BLK), jnp.int32),
            pltpu.VMEM((_NBUF, BLK, w), F32),
            pltpu.VMEM((zrows, w), F32),
            pltpu.VMEM_SHARED((nagg, w), F32),
            [pltpu.SemaphoreType.DMA] * _NBUF,
            [pltpu.SemaphoreType.DMA] * _NBUF,
        ],
        compiler_params=pltpu.CompilerParams(use_tc_tiling_on_sc=False),
    )
    def edge_pass(t_hbm, src_hbm, dst_hbm, z_hbm, agg_hbm,
                  src_v, dst_v, rows_v, zb_v, agg_sh, gsems, ssems):
        c = lax.axis_index("c")
        s = lax.axis_index("s")
        wid = c * NS + s
        pltpu.sync_copy(src_hbm.at[pl.ds(wid * bpt, bpt)], src_v)
        pltpu.sync_copy(dst_hbm.at[pl.ds(wid * bpt, bpt)], dst_v)
        pltpu.sync_copy(z_hbm, zb_v)
        pltpu.sync_copy(zb_v, agg_sh.at[pl.ds(s * zrows, zrows)])
        plsc.subcore_barrier()

        for j in range(_NBUF):
            pltpu.async_copy(t_hbm.at[src_v.at[j]], rows_v.at[j], gsems[j])

        def body(g, carry):
            base = g * _NBUF
            for j in range(_NBUF):
                b = base + j
                pltpu.make_async_copy(
                    t_hbm.at[src_v.at[b]], rows_v.at[j], gsems[j]).wait()
                pltpu.sync_copy(rows_v.at[j], agg_sh.at[dst_v.at[b]], add=True)

                @pl.when(b + _NBUF < bpt)
                def _():
                    pltpu.async_copy(
                        t_hbm.at[src_v.at[b + _NBUF]], rows_v.at[j], gsems[j])
            return carry

        lax.fori_loop(0, bpt // _NBUF, body, 0)
        plsc.subcore_barrier()
        pltpu.sync_copy(agg_sh.at[pl.ds(s * zrows, zrows)],
                        agg_hbm.at[c, pl.ds(s * zrows, zrows)])

    return edge_pass


def kernel(x, edge_index, batch, W1_rel, b1, W1_root, W2_rel, b2, W2_root,
           W3_rel, b3, W3_root, W_lin1, b_lin1, W_lin2, b_lin2):
    n, d = x.shape
    e = edge_index.shape[1]
    h = W1_rel.shape[1]
    g = 64
    per_pass = BLK * NC * NS * 8
    epad = -(-e // per_pass) * per_pass
    nblk = epad // BLK
    trash = n

    src = edge_index[0]
    dst = edge_index[1]
    pad = epad - e
    nagg = (n // (NS * 8) + 1) * NS * 8
    pad_dst = trash + jnp.arange(pad, dtype=jnp.int32) % (nagg - n)
    srcb = jnp.concatenate([src, jnp.zeros((pad,), jnp.int32)]).reshape(nblk, BLK)
    dstb = jnp.concatenate([dst, pad_dst]).reshape(nblk, BLK)
    z16 = jnp.zeros((nagg // NS, 16), F32)
    z8 = jnp.zeros((nagg // NS, h), F32)

    edge_pass16 = _make_edge_pass(n, nblk, 16)
    edge_pass8 = _make_edge_pass(n, nblk, h)

    proj = pl.pallas_call(
        _proj_body,
        out_shape=[jax.ShapeDtypeStruct((n, 16), F32),
                   jax.ShapeDtypeStruct((n, h), F32)])
    layer1 = pl.pallas_call(
        _layer1_body,
        out_shape=[jax.ShapeDtypeStruct((n, h), F32),
                   jax.ShapeDtypeStruct((n, h), F32),
                   jax.ShapeDtypeStruct((n, 1), F32)])
    layer2 = pl.pallas_call(
        _layer2_body,
        out_shape=[jax.ShapeDtypeStruct((n, h), F32),
                   jax.ShapeDtypeStruct((n, h), F32)])
    final = pl.pallas_call(
        _final_body,
        out_shape=jax.ShapeDtypeStruct((g, 1), F32))

    t1, r1 = proj(x, W1_rel, W1_root)
    agg1 = edge_pass16(t1, srcb, dstb, z16)
    t2, r2, degr = layer1(agg1, r1, b1.reshape(1, h), W2_rel, W2_root)
    agg2 = edge_pass8(t2, srcb, dstb, z8)
    t3, r3 = layer2(agg2, r2, degr, b2.reshape(1, h), W3_rel, W3_root)
    agg3 = edge_pass8(t3, srcb, dstb, z8)
    out = final(agg3, r3, degr, b3.reshape(1, h),
                batch.reshape(1, n).astype(jnp.int32),
                W_lin1, b_lin1.reshape(1, -1), W_lin2, b_lin2.reshape(1, -1))
    return out


# speedup vs baseline: 1.6777x; 1.5864x over previous
import functools

import jax
import jax.numpy as jnp
from jax import lax
from jax.experimental import pallas as pl
from jax.experimental.pallas import tpu as pltpu
from jax.experimental.pallas import tpu_sc as plsc

F32 = jnp.float32

NC = 2
NS = 16
BLK = 256


def _proj_body(x_ref, wrel_ref, wroot_ref, t_ref, r_ref):
    x = x_ref[...]
    n = x.shape[0]
    p = jnp.dot(x, wrel_ref[...], preferred_element_type=F32)
    hi = (lax.broadcasted_iota(jnp.int32, (n, 8), 1) == 0).astype(F32)
    t_ref[:n, :] = jnp.concatenate([p, hi], axis=1)
    r_ref[...] = jnp.dot(x, wroot_ref[...], preferred_element_type=F32)


def _layer1_body(agg_ref, r_ref, b_ref, wrel_ref, wroot_ref,
                 t_ref, rout_ref, degr_ref):
    nn = r_ref.shape[0]
    a = (agg_ref[0] + agg_ref[1])[:nn]
    degr_ref[...] = 1.0 / jnp.maximum(a[:, 8:9], 1.0)
    h = jnp.maximum(a[:, 0:8] + b_ref[...] + r_ref[...], 0.0)
    t_ref[:nn, :] = jnp.dot(h, wrel_ref[...], preferred_element_type=F32)
    rout_ref[...] = jnp.dot(h, wroot_ref[...], preferred_element_type=F32)


def _layer2_body(agg_ref, r_ref, degr_ref, b_ref, wrel_ref, wroot_ref,
                 t_ref, rout_ref):
    nn = r_ref.shape[0]
    a = (agg_ref[0] + agg_ref[1])[:nn]
    h = jnp.maximum(a[:, 0:8] * degr_ref[...] + b_ref[...] + r_ref[...], 0.0)
    t_ref[:nn, :] = jnp.dot(h, wrel_ref[...], preferred_element_type=F32)
    rout_ref[...] = jnp.dot(h, wroot_ref[...], preferred_element_type=F32)


def _final_body(agg_ref, r_ref, degr_ref, b_ref, batch_ref,
                wl1_ref, bl1_ref, wl2_ref, bl2_ref, out_ref):
    nn = r_ref.shape[0]
    a = (agg_ref[0] + agg_ref[1])[:nn]
    h = jnp.maximum(a[:, 0:8] * degr_ref[...] + b_ref[...] + r_ref[...], 0.0)
    n = h.shape[0]
    g = out_ref.shape[0]
    onehot = (batch_ref[...] ==
              lax.broadcasted_iota(jnp.int32, (g, n), 0)).astype(F32)
    pooled_sum = jnp.dot(onehot, h, preferred_element_type=F32)
    cnt = jnp.maximum(jnp.sum(onehot, axis=1, keepdims=True), 1.0)
    pooled = pooled_sum / cnt
    z = jnp.maximum(
        jnp.dot(pooled, wl1_ref[...], preferred_element_type=F32) + bl1_ref[...],
        0.0)
    out_ref[...] = (jnp.dot(z, wl2_ref[...], preferred_element_type=F32)
                    + bl2_ref[...])


_NBUF = 4


def _make_edge_pass(n, nblk, w):
    bpt = nblk // (NC * NS)
    nagg = (n // (NS * 8) + 1) * NS * 8
    zrows = nagg // NS
    mesh = plsc.VectorSubcoreMesh(core_axis_name="c", subcore_axis_name="s",
                                  num_cores=NC, num_subcores=NS)

    @functools.partial(
        pl.kernel,
        out_type=jax.ShapeDtypeStruct((NC, nagg, w), F32),
        mesh=mesh,
        scratch_types=[
            pltpu.VMEM((bpt, BLK), jnp.int32),
            pltpu.VMEM((bpt, BLK), jnp.int32),
            pltpu.VMEM((_NBUF, BLK, w), F32),
            pltpu.VMEM((zrows, w), F32),
            pltpu.VMEM_SHARED((nagg, w), F32),
            pltpu.VMEM_SHARED((nagg, w), F32),
            [pltpu.SemaphoreType.DMA] * _NBUF,
            [pltpu.SemaphoreType.DMA] * _NBUF,
        ],
        compiler_params=pltpu.CompilerParams(use_tc_tiling_on_sc=False),
    )
    def edge_pass(t_hbm, src_hbm, dst_hbm, z_hbm, agg_hbm,
                  src_v, dst_v, rows_v, zb_v, agg_sh, tab_sh, gsems, ssems):
        c = lax.axis_index("c")
        s = lax.axis_index("s")
        wid = c * NS + s
        pltpu.sync_copy(src_hbm.at[pl.ds(wid * bpt, bpt)], src_v)
        pltpu.sync_copy(dst_hbm.at[pl.ds(wid * bpt, bpt)], dst_v)
        pltpu.sync_copy(t_hbm.at[pl.ds(s * zrows, zrows)],
                        tab_sh.at[pl.ds(s * zrows, zrows)])
        pltpu.sync_copy(z_hbm, zb_v)
        pltpu.sync_copy(zb_v, agg_sh.at[pl.ds(s * zrows, zrows)])
        plsc.subcore_barrier()

        for j in range(_NBUF):
            pltpu.async_copy(tab_sh.at[src_v.at[j]], rows_v.at[j], gsems[j])

        def body(g, carry):
            base = g * _NBUF
            for j in range(_NBUF):
                b = base + j
                pltpu.make_async_copy(
                    tab_sh.at[src_v.at[b]], rows_v.at[j], gsems[j]).wait()
                pltpu.sync_copy(rows_v.at[j], agg_sh.at[dst_v.at[b]], add=True)

                @pl.when(b + _NBUF < bpt)
                def _():
                    pltpu.async_copy(
                        tab_sh.at[src_v.at[b + _NBUF]], rows_v.at[j], gsems[j])
            return carry

        lax.fori_loop(0, bpt // _NBUF, body, 0)
        plsc.subcore_barrier()
        pltpu.sync_copy(agg_sh.at[pl.ds(s * zrows, zrows)],
                        agg_hbm.at[c, pl.ds(s * zrows, zrows)])

    return edge_pass


def kernel(x, edge_index, batch, W1_rel, b1, W1_root, W2_rel, b2, W2_root,
           W3_rel, b3, W3_root, W_lin1, b_lin1, W_lin2, b_lin2):
    n, d = x.shape
    e = edge_index.shape[1]
    h = W1_rel.shape[1]
    g = 64
    per_pass = BLK * NC * NS * 8
    epad = -(-e // per_pass) * per_pass
    nblk = epad // BLK
    trash = n

    src = edge_index[0]
    dst = edge_index[1]
    pad = epad - e
    nagg = (n // (NS * 8) + 1) * NS * 8
    pad_dst = trash + jnp.arange(pad, dtype=jnp.int32) % (nagg - n)
    srcb = jnp.concatenate([src, jnp.zeros((pad,), jnp.int32)]).reshape(nblk, BLK)
    dstb = jnp.concatenate([dst, pad_dst]).reshape(nblk, BLK)
    z16 = jnp.zeros((nagg // NS, 16), F32)
    z8 = jnp.zeros((nagg // NS, h), F32)

    edge_pass16 = _make_edge_pass(n, nblk, 16)
    edge_pass8 = _make_edge_pass(n, nblk, h)

    proj = pl.pallas_call(
        _proj_body,
        out_shape=[jax.ShapeDtypeStruct((nagg, 16), F32),
                   jax.ShapeDtypeStruct((n, h), F32)])
    layer1 = pl.pallas_call(
        _layer1_body,
        out_shape=[jax.ShapeDtypeStruct((nagg, h), F32),
                   jax.ShapeDtypeStruct((n, h), F32),
                   jax.ShapeDtypeStruct((n, 1), F32)])
    layer2 = pl.pallas_call(
        _layer2_body,
        out_shape=[jax.ShapeDtypeStruct((nagg, h), F32),
                   jax.ShapeDtypeStruct((n, h), F32)])
    final = pl.pallas_call(
        _final_body,
        out_shape=jax.ShapeDtypeStruct((g, 1), F32))

    t1, r1 = proj(x, W1_rel, W1_root)
    agg1 = edge_pass16(t1, srcb, dstb, z16)
    t2, r2, degr = layer1(agg1, r1, b1.reshape(1, h), W2_rel, W2_root)
    agg2 = edge_pass8(t2, srcb, dstb, z8)
    t3, r3 = layer2(agg2, r2, degr, b2.reshape(1, h), W3_rel, W3_root)
    agg3 = edge_pass8(t3, srcb, dstb, z8)
    out = final(agg3, r3, degr, b3.reshape(1, h),
                batch.reshape(1, n).astype(jnp.int32),
                W_lin1, b_lin1.reshape(1, -1), W_lin2, b_lin2.reshape(1, -1))
    return out
